# baseline (speedup 1.0000x reference)
import functools

import jax
import jax.numpy as jnp
from jax import lax
from jax.experimental import pallas as pl
from jax.experimental.pallas import tpu as pltpu

_U = jnp.uint32


def _threefry2x32(ks0, ks1, x0, x1):
    ks2 = ks0 ^ ks1 ^ 0x1BD11BDA
    rot0 = (13, 15, 26, 6)
    rot1 = (17, 29, 16, 24)

    def rounds(a, b, rots):
        for r in rots:
            a = a + b
            b = (b << _U(r)) | (b >> _U(32 - r))
            b = a ^ b
        return a, b

    x0 = x0 + _U(ks0)
    x1 = x1 + _U(ks1)
    x0, x1 = rounds(x0, x1, rot0)
    x0 = x0 + _U(ks1)
    x1 = x1 + _U(ks2 + 1)
    x0, x1 = rounds(x0, x1, rot1)
    x0 = x0 + _U(ks2)
    x1 = x1 + _U(ks0 + 2)
    x0, x1 = rounds(x0, x1, rot0)
    x0 = x0 + _U(ks0)
    x1 = x1 + _U(ks1 + 3)
    x0, x1 = rounds(x0, x1, rot1)
    x0 = x0 + _U(ks1)
    x1 = x1 + _U(ks2 + 4)
    x0, x1 = rounds(x0, x1, rot0)
    x0 = x0 + _U(ks2)
    x1 = x1 + _U(ks0 + 5)
    return x0, x1


def _body(x_ref, lp_ref, sel_ref, act_ref, *, n_cols, row_block, n_types,
          per_type, key_hi, key_lo):
    pid = pl.program_id(0)
    x = x_ref[...]

    m = jnp.max(x, axis=1, keepdims=True)
    shifted = x - m
    s = jnp.sum(jnp.exp(shifted), axis=1, keepdims=True)
    lp = shifted - jnp.log(s)
    lp_ref[...] = lp

    col_u = lax.broadcasted_iota(_U, (row_block, n_cols), 1)
    row_u = lax.broadcasted_iota(_U, (row_block, n_cols), 0)
    flat = (_U(row_block) * pid.astype(_U) + row_u) * _U(n_cols) + col_u
    b0, b1 = _threefry2x32(key_hi, key_lo, jnp.zeros_like(flat), flat)
    bits = b0 ^ b1
    tiny = jnp.float32(jnp.finfo(jnp.float32).tiny)
    fbits = (bits >> _U(9)) | _U(0x3F800000)
    fl = lax.bitcast_convert_type(fbits, jnp.float32) - jnp.float32(1.0)
    u = lax.max(tiny, fl * (jnp.float32(1.0) - tiny) + tiny)
    g = -jnp.log(-jnp.log(u))

    p = lp + g
    pm = jnp.max(p, axis=1, keepdims=True)
    col_i = lax.broadcasted_iota(jnp.int32, (row_block, n_cols), 1)
    idx = jnp.min(jnp.where(p == pm, col_i, jnp.int32(n_cols)), axis=1,
                  keepdims=True)
    neg_inf = jnp.float32(-jnp.inf)
    sel = jnp.max(jnp.where(col_i == idx, lp, neg_inf), axis=1, keepdims=True)
    sel_ref[...] = sel

    ty = jnp.zeros((row_block, 1), jnp.int32)
    for t in range(1, n_types):
        ty = ty + jnp.where(idx >= t * per_type, 1, 0).astype(jnp.int32)
    pa = idx - ty * jnp.int32(per_type)
    act_ref[...] = jnp.concatenate([ty, pa], axis=1)


def _run(logits, *, n_types, per_type, key_hi, key_lo, row_block=8,
         interpret=False):
    b, n = logits.shape
    grid = (b // row_block,)
    body = functools.partial(_body, n_cols=n, row_block=row_block,
                             n_types=n_types, per_type=per_type,
                             key_hi=key_hi, key_lo=key_lo)
    lp, sel, act = pl.pallas_call(
        body,
        grid=grid,
        in_specs=[pl.BlockSpec((row_block, n), lambda g: (g, 0))],
        out_specs=[
            pl.BlockSpec((row_block, n), lambda g: (g, 0)),
            pl.BlockSpec((row_block, 1), lambda g: (g, 0)),
            pl.BlockSpec((row_block, 2), lambda g: (g, 0)),
        ],
        out_shape=[
            jax.ShapeDtypeStruct((b, n), jnp.float32),
            jax.ShapeDtypeStruct((b, 1), jnp.float32),
            jax.ShapeDtypeStruct((b, 2), jnp.int32),
        ],
        compiler_params=pltpu.CompilerParams(
            dimension_semantics=("arbitrary",)),
        interpret=interpret,
    )(logits)
    return act, sel[:, 0], lp


def kernel(logits, action_index_tensor):
    del action_index_tensor
    return _run(logits, n_types=10, per_type=10000, key_hi=0, key_lo=42)
